# baseline (speedup 1.0000x reference)
import functools

import numpy as np
import jax
import jax.numpy as jnp
from jax import lax
from jax.experimental import pallas as pl
from jax.experimental.pallas import tpu as pltpu
from jax.experimental.pallas import tpu_sc as plsc

N = 10000
E = 320000
F = 128
R = 32
NR = 3

NB = 1000
EB = 8000

NC = 2
NS = 16
L = 16
NW = NC * NS
EPW = E // NW
C = 80
NCHUNK = EPW // C
NPAD = 10240
TPN = NPAD // NS
TCH = 32


def _sp(x):
    return jnp.maximum(x, 0.0) + jnp.log1p(jnp.exp(-jnp.abs(x)))


def _node_pre_body(x_ref, wi_ref, bi_ref, wj_ref, bj_ref, e_ref, u0_ref, ej_ref):
    e = _sp(x_ref[...])
    e_ref[...] = e
    u0_ref[...] = _sp(jnp.dot(e, wi_ref[...], preferred_element_type=jnp.float32) + bi_ref[...])
    ej_ref[...] = _sp(jnp.dot(e, wj_ref[...], preferred_element_type=jnp.float32) + bj_ref[...])


_node_pre = pl.pallas_call(
    _node_pre_body,
    grid=(N // NB,),
    in_specs=[
        pl.BlockSpec((NB, F), lambda i: (i, 0)),
        pl.BlockSpec((F, F), lambda i: (0, 0)),
        pl.BlockSpec((1, F), lambda i: (0, 0)),
        pl.BlockSpec((F, F), lambda i: (0, 0)),
        pl.BlockSpec((1, F), lambda i: (0, 0)),
    ],
    out_specs=[pl.BlockSpec((NB, F), lambda i: (i, 0))] * 3,
    out_shape=[jax.ShapeDtypeStruct((N, F), jnp.float32)] * 3,
)


_PERM = np.empty((F,), dtype=np.int32)
for _gi in range(F // 32):
    for _li in range(16):
        _PERM[32 * _gi + 2 * _li] = 32 * _gi + _li
        _PERM[32 * _gi + 2 * _li + 1] = 32 * _gi + 16 + _li


def _g_body(f_ref, wg_ref, g_ref):
    g_ref[...] = jnp.dot(f_ref[...], wg_ref[...],
                         preferred_element_type=jnp.float32).astype(jnp.bfloat16)


_g_mat = pl.pallas_call(
    _g_body,
    grid=(E // EB,),
    in_specs=[
        pl.BlockSpec((EB, R), lambda i: (i, 0)),
        pl.BlockSpec((R, F), lambda i: (0, 0)),
    ],
    out_specs=pl.BlockSpec((EB, F), lambda i: (i, 0)),
    out_shape=jax.ShapeDtypeStruct((E, F), jnp.bfloat16),
)


@functools.partial(
    pl.kernel,
    out_type=jax.ShapeDtypeStruct((NC, NPAD, F), jnp.float32),
    mesh=plsc.VectorSubcoreMesh(core_axis_name="c", subcore_axis_name="s"),
    scratch_types=[
        pltpu.VMEM((C,), jnp.int32),
        pltpu.VMEM((C,), jnp.int32),
        pltpu.VMEM((C,), jnp.int32),
        pltpu.VMEM((C,), jnp.int32),
        pltpu.VMEM((C, F), jnp.float32),
        pltpu.VMEM((C, F), jnp.float32),
        pltpu.VMEM((C, F // 2), jnp.int32),
        pltpu.VMEM((C, F // 2), jnp.int32),
        pltpu.VMEM((C,), jnp.int32),
        pltpu.VMEM((C,), jnp.int32),
        pltpu.VMEM((TCH, F), jnp.float32),
        pltpu.VMEM_SHARED((NPAD, F), jnp.float32),
        pltpu.SemaphoreType.DMA,
        pltpu.SemaphoreType.DMA,
        pltpu.SemaphoreType.DMA,
        pltpu.SemaphoreType.DMA,
        pltpu.SemaphoreType.DMA,
        pltpu.SemaphoreType.DMA,
        pltpu.SemaphoreType.DMA,
        pltpu.SemaphoreType.DMA,
        pltpu.SemaphoreType.DMA,
        pltpu.SemaphoreType.DMA,
    ],
)
def _edge_kernel(ii_hbm, jj_hbm, g_hbm, ej_hbm, out_hbm,
                 ii0, ii1, jj0, jj1, rows0, rows1, g0, g1, sii0, sii1,
                 stage_v, acc_sh,
                 sem_ii0, sem_ii1, sem_jj0, sem_jj1, sem_ga0, sem_ga1,
                 sem_gl0, sem_gl1, sem_s0, sem_s1):
    iiv = (ii0, ii1)
    jjv = (jj0, jj1)
    rows = (rows0, rows1)
    gv = (g0, g1)
    sii = (sii0, sii1)
    sem_ii = (sem_ii0, sem_ii1)
    sem_jj = (sem_jj0, sem_jj1)
    sem_ga = (sem_ga0, sem_ga1)
    sem_gl = (sem_gl0, sem_gl1)
    sem_s = (sem_s0, sem_s1)

    c = lax.axis_index("c")
    s = lax.axis_index("s")
    wid = s * NC + c
    base_w = wid * EPW

    zv = jnp.zeros((L,), jnp.float32)

    def _zrow(i, cc):
        for v in range(F // L):
            stage_v[i, pl.ds(v * L, L)] = zv
        return cc

    lax.fori_loop(0, TCH, _zrow, 0)
    for t in range(TPN // TCH):
        pltpu.sync_copy(stage_v, acc_sh.at[pl.ds(s * TPN + t * TCH, TCH)])
    plsc.subcore_barrier()

    def start_I(k, b):
        base = base_w + k * C
        pltpu.async_copy(ii_hbm.at[pl.ds(base, C)], iiv[b], sem_ii[b])
        pltpu.async_copy(jj_hbm.at[pl.ds(base, C)], jjv[b], sem_jj[b])

    def wait_ii(b):
        pltpu.make_async_copy(ii_hbm.at[pl.ds(0, C)], iiv[b], sem_ii[b]).wait()

    def wait_jj(b):
        pltpu.make_async_copy(jj_hbm.at[pl.ds(0, C)], jjv[b], sem_jj[b]).wait()

    def start_G(k, b):
        base = base_w + k * C
        pltpu.async_copy(ej_hbm.at[jjv[b]], rows[b], sem_ga[b])
        pltpu.async_copy(g_hbm.at[pl.ds(base, C)], gv[b], sem_gl[b])

    def wait_G(b):
        pltpu.make_async_copy(ej_hbm.at[pl.ds(0, C)], rows[b], sem_ga[b]).wait()
        pltpu.make_async_copy(g_hbm.at[pl.ds(0, C)], gv[b], sem_gl[b]).wait()

    def copy_sii(b):
        for v in range(C // L):
            sii[b][pl.ds(v * L, L)] = iiv[b][pl.ds(v * L, L)]

    def mul(b):
        def _mrow(i, cc):
            for v in range(F // 32):
                gi = gv[b][i, pl.ds(L * v, L)]
                glo = jax.lax.bitcast_convert_type(
                    jax.lax.shift_left(gi, 16), jnp.float32)
                ghi = jax.lax.bitcast_convert_type(
                    jax.lax.bitwise_and(gi, jnp.int32(-65536)), jnp.float32)
                ix_lo = (i, pl.ds(32 * v, L))
                ix_hi = (i, pl.ds(32 * v + L, L))
                rows[b][ix_lo] = rows[b][ix_lo] * glo
                rows[b][ix_hi] = rows[b][ix_hi] * ghi
            return cc

        lax.fori_loop(0, C, _mrow, 0)

    def start_S(b):
        pltpu.async_copy(rows[b], acc_sh.at[sii[b]], sem_s[b], add=True)

    def wait_S(b):
        pltpu.make_async_copy(ej_hbm.at[pl.ds(0, C)], rows[b], sem_s[b]).wait()

    def step(k, b):
        wait_G(b)
        wait_ii(b)
        copy_sii(b)
        start_I(k + 2, b)
        wait_S(1 - b)
        wait_jj(1 - b)
        start_G(k + 1, 1 - b)
        mul(b)
        start_S(b)

    start_I(0, 0)
    start_I(1, 1)
    wait_jj(0)
    start_G(0, 0)
    wait_G(0)
    wait_ii(0)
    copy_sii(0)
    start_I(2, 0)
    wait_jj(1)
    start_G(1, 1)
    mul(0)
    start_S(0)

    def _pair(j2, cc):
        k = 1 + 2 * j2
        step(k, 1)
        step(k + 1, 0)
        return cc

    lax.fori_loop(0, (NCHUNK - 3) // 2, _pair, 0)

    wait_G(1)
    wait_ii(1)
    copy_sii(1)
    wait_S(0)
    wait_jj(0)
    start_G(NCHUNK - 1, 0)
    mul(1)
    start_S(1)

    wait_G(0)
    wait_ii(0)
    wait_S(1)
    mul(0)
    pltpu.sync_copy(rows0, acc_sh.at[ii0], add=True)

    plsc.subcore_barrier()

    for t in range(TPN // TCH):
        pltpu.sync_copy(acc_sh.at[pl.ds(s * TPN + t * TCH, TCH)], stage_v)
        pltpu.sync_copy(stage_v, out_hbm.at[c, pl.ds(s * TPN + t * TCH, TCH)])


def _post_body(u0_ref, a_ref, e_ref, wr1_ref, br1_ref, wr2_ref, br2_ref,
               wv_ref, bv_ref, gate_ref, out_ref):
    u = u0_ref[...] + a_ref[0] + a_ref[1]
    for k in range(NR):
        h = _sp(u)
        h = _sp(jnp.dot(h, wr1_ref[k], preferred_element_type=jnp.float32) + br1_ref[k])
        h = jnp.dot(h, wr2_ref[k], preferred_element_type=jnp.float32) + br2_ref[k]
        u = u + h
    u = _sp(u)
    out_ref[...] = (gate_ref[...] * e_ref[...]
                    + jnp.dot(u, wv_ref[...], preferred_element_type=jnp.float32)
                    + bv_ref[...])


_post = pl.pallas_call(
    _post_body,
    grid=(N // NB,),
    in_specs=[
        pl.BlockSpec((NB, F), lambda i: (i, 0)),
        pl.BlockSpec((NC, NB, F), lambda i: (0, i, 0)),
        pl.BlockSpec((NB, F), lambda i: (i, 0)),
        pl.BlockSpec((NR, F, F), lambda i: (0, 0, 0)),
        pl.BlockSpec((NR, 1, F), lambda i: (0, 0, 0)),
        pl.BlockSpec((NR, F, F), lambda i: (0, 0, 0)),
        pl.BlockSpec((NR, 1, F), lambda i: (0, 0, 0)),
        pl.BlockSpec((F, F), lambda i: (0, 0)),
        pl.BlockSpec((1, F), lambda i: (0, 0)),
        pl.BlockSpec((1, F), lambda i: (0, 0)),
    ],
    out_specs=pl.BlockSpec((NB, F), lambda i: (i, 0)),
    out_shape=jax.ShapeDtypeStruct((N, F), jnp.float32),
)


def kernel(pair_indices, atomic_embedding, f_ij, W_g, W_i, b_i, W_j, b_j,
           W_v, b_v, Wr1, br1, Wr2, br2, gate):
    idx = pair_indices.astype(jnp.int32)
    idx_i = idx[0]
    idx_j = idx[1]

    e, u0, ej = _node_pre(atomic_embedding, W_i, b_i.reshape(1, F), W_j, b_j.reshape(1, F))
    g = _g_mat(f_ij, W_g[:, _PERM])
    g_i32 = jax.lax.bitcast_convert_type(g.reshape(E, F // 2, 2), jnp.int32)
    accs = _edge_kernel(idx_i, idx_j, g_i32, ej)
    out = _post(u0, accs, e, Wr1, br1.reshape(NR, 1, F), Wr2, br2.reshape(NR, 1, F),
                W_v, b_v.reshape(1, F), gate.reshape(1, F))
    return out


# speedup vs baseline: 2.7740x; 2.7740x over previous
import functools

import numpy as np
import jax
import jax.numpy as jnp
from jax import lax
from jax.experimental import pallas as pl
from jax.experimental.pallas import tpu as pltpu
from jax.experimental.pallas import tpu_sc as plsc

N = 10000
E = 320000
F = 128
R = 32
NR = 3

NB = 1000
EB = 8000

NC = 2
NS = 16
L = 16
NW = NC * NS
EPW = E // NW
C = 80
NCHUNK = EPW // C
NPAD = 10240
TPN = NPAD // NS
TCH = 32


def _sp(x):
    return jnp.maximum(x, 0.0) + jnp.log1p(jnp.exp(-jnp.abs(x)))


def _node_pre_body(x_ref, wi_ref, bi_ref, wj_ref, bj_ref, e_ref, u0_ref, ej_ref):
    e = _sp(x_ref[...])
    e_ref[...] = e
    u0_ref[...] = _sp(jnp.dot(e, wi_ref[...], preferred_element_type=jnp.float32) + bi_ref[...])
    ej_ref[...] = _sp(jnp.dot(e, wj_ref[...], preferred_element_type=jnp.float32) + bj_ref[...])


_node_pre = pl.pallas_call(
    _node_pre_body,
    grid=(N // NB,),
    in_specs=[
        pl.BlockSpec((NB, F), lambda i: (i, 0)),
        pl.BlockSpec((F, F), lambda i: (0, 0)),
        pl.BlockSpec((1, F), lambda i: (0, 0)),
        pl.BlockSpec((F, F), lambda i: (0, 0)),
        pl.BlockSpec((1, F), lambda i: (0, 0)),
    ],
    out_specs=[pl.BlockSpec((NB, F), lambda i: (i, 0))] * 3,
    out_shape=[jax.ShapeDtypeStruct((N, F), jnp.float32)] * 3,
)


def _g_body(f_ref, wg_ref, g_ref):
    x = jnp.dot(f_ref[...], wg_ref[...], preferred_element_type=jnp.float32)
    y = jax.lax.bitcast_convert_type(x, jnp.int32)
    r = y + jnp.int32(0x7FFF) + jax.lax.bitwise_and(
        jax.lax.shift_right_logical(y, 16), jnp.int32(1))
    lo = jax.lax.bitwise_and(jax.lax.shift_right_logical(r[:, :F // 2], 16),
                             jnp.int32(0xFFFF))
    hi = jax.lax.bitwise_and(r[:, F // 2:], jnp.int32(-65536))
    g_ref[...] = jax.lax.bitwise_or(lo, hi)


_g_mat = pl.pallas_call(
    _g_body,
    grid=(E // EB,),
    in_specs=[
        pl.BlockSpec((EB, R), lambda i: (i, 0)),
        pl.BlockSpec((R, F), lambda i: (0, 0)),
    ],
    out_specs=pl.BlockSpec((EB, F // 2), lambda i: (i, 0)),
    out_shape=jax.ShapeDtypeStruct((E, F // 2), jnp.int32),
)


@functools.partial(
    pl.kernel,
    out_type=jax.ShapeDtypeStruct((NC, NPAD, F), jnp.float32),
    mesh=plsc.VectorSubcoreMesh(core_axis_name="c", subcore_axis_name="s"),
    scratch_types=[
        pltpu.VMEM((C,), jnp.int32),
        pltpu.VMEM((C,), jnp.int32),
        pltpu.VMEM((C,), jnp.int32),
        pltpu.VMEM((C,), jnp.int32),
        pltpu.VMEM((C, F), jnp.float32),
        pltpu.VMEM((C, F), jnp.float32),
        pltpu.VMEM((C, F // 2), jnp.int32),
        pltpu.VMEM((C, F // 2), jnp.int32),
        pltpu.VMEM((C,), jnp.int32),
        pltpu.VMEM((C,), jnp.int32),
        pltpu.VMEM((TCH, F), jnp.float32),
        pltpu.VMEM_SHARED((NPAD, F), jnp.float32),
        pltpu.SemaphoreType.DMA,
        pltpu.SemaphoreType.DMA,
        pltpu.SemaphoreType.DMA,
        pltpu.SemaphoreType.DMA,
        pltpu.SemaphoreType.DMA,
        pltpu.SemaphoreType.DMA,
        pltpu.SemaphoreType.DMA,
        pltpu.SemaphoreType.DMA,
        pltpu.SemaphoreType.DMA,
        pltpu.SemaphoreType.DMA,
    ],
)
def _edge_kernel(ii_hbm, jj_hbm, g_hbm, ej_hbm, out_hbm,
                 ii0, ii1, jj0, jj1, rows0, rows1, g0, g1, sii0, sii1,
                 stage_v, acc_sh,
                 sem_ii0, sem_ii1, sem_jj0, sem_jj1, sem_ga0, sem_ga1,
                 sem_gl0, sem_gl1, sem_s0, sem_s1):
    iiv = (ii0, ii1)
    jjv = (jj0, jj1)
    rows = (rows0, rows1)
    gv = (g0, g1)
    sii = (sii0, sii1)
    sem_ii = (sem_ii0, sem_ii1)
    sem_jj = (sem_jj0, sem_jj1)
    sem_ga = (sem_ga0, sem_ga1)
    sem_gl = (sem_gl0, sem_gl1)
    sem_s = (sem_s0, sem_s1)

    c = lax.axis_index("c")
    s = lax.axis_index("s")
    wid = s * NC + c
    base_w = wid * EPW

    zv = jnp.zeros((L,), jnp.float32)

    def _zrow(i, cc):
        for v in range(F // L):
            stage_v[i, pl.ds(v * L, L)] = zv
        return cc

    lax.fori_loop(0, TCH, _zrow, 0)
    for t in range(TPN // TCH):
        pltpu.sync_copy(stage_v, acc_sh.at[pl.ds(s * TPN + t * TCH, TCH)])
    plsc.subcore_barrier()

    def start_I(k, b):
        base = base_w + k * C
        pltpu.async_copy(ii_hbm.at[pl.ds(base, C)], iiv[b], sem_ii[b])
        pltpu.async_copy(jj_hbm.at[pl.ds(base, C)], jjv[b], sem_jj[b])

    def wait_ii(b):
        pltpu.make_async_copy(ii_hbm.at[pl.ds(0, C)], iiv[b], sem_ii[b]).wait()

    def wait_jj(b):
        pltpu.make_async_copy(jj_hbm.at[pl.ds(0, C)], jjv[b], sem_jj[b]).wait()

    def start_G(k, b):
        base = base_w + k * C
        pltpu.async_copy(ej_hbm.at[jjv[b]], rows[b], sem_ga[b])
        pltpu.async_copy(g_hbm.at[pl.ds(base, C)], gv[b], sem_gl[b])

    def wait_G(b):
        pltpu.make_async_copy(ej_hbm.at[pl.ds(0, C)], rows[b], sem_ga[b]).wait()
        pltpu.make_async_copy(g_hbm.at[pl.ds(0, C)], gv[b], sem_gl[b]).wait()

    def copy_sii(b):
        for v in range(C // L):
            sii[b][pl.ds(v * L, L)] = iiv[b][pl.ds(v * L, L)]

    def mul(b):
        def _mrow(i, cc):
            for v in range(F // 32):
                gi = gv[b][i, pl.ds(L * v, L)]
                glo = jax.lax.bitcast_convert_type(
                    jax.lax.shift_left(gi, 16), jnp.float32)
                ghi = jax.lax.bitcast_convert_type(
                    jax.lax.bitwise_and(gi, jnp.int32(-65536)), jnp.float32)
                ix_lo = (i, pl.ds(L * v, L))
                ix_hi = (i, pl.ds(F // 2 + L * v, L))
                rows[b][ix_lo] = rows[b][ix_lo] * glo
                rows[b][ix_hi] = rows[b][ix_hi] * ghi
            return cc

        lax.fori_loop(0, C, _mrow, 0)

    def start_S(b):
        pltpu.async_copy(rows[b], acc_sh.at[sii[b]], sem_s[b], add=True)

    def wait_S(b):
        pltpu.make_async_copy(ej_hbm.at[pl.ds(0, C)], rows[b], sem_s[b]).wait()

    def step(k, b):
        wait_G(b)
        wait_ii(b)
        copy_sii(b)
        start_I(k + 2, b)
        wait_S(1 - b)
        wait_jj(1 - b)
        start_G(k + 1, 1 - b)
        mul(b)
        start_S(b)

    start_I(0, 0)
    start_I(1, 1)
    wait_jj(0)
    start_G(0, 0)
    wait_G(0)
    wait_ii(0)
    copy_sii(0)
    start_I(2, 0)
    wait_jj(1)
    start_G(1, 1)
    mul(0)
    start_S(0)

    def _pair(j2, cc):
        k = 1 + 2 * j2
        step(k, 1)
        step(k + 1, 0)
        return cc

    lax.fori_loop(0, (NCHUNK - 3) // 2, _pair, 0)

    wait_G(1)
    wait_ii(1)
    copy_sii(1)
    wait_S(0)
    wait_jj(0)
    start_G(NCHUNK - 1, 0)
    mul(1)
    start_S(1)

    wait_G(0)
    wait_ii(0)
    wait_S(1)
    mul(0)
    pltpu.sync_copy(rows0, acc_sh.at[ii0], add=True)

    plsc.subcore_barrier()

    for t in range(TPN // TCH):
        pltpu.sync_copy(acc_sh.at[pl.ds(s * TPN + t * TCH, TCH)], stage_v)
        pltpu.sync_copy(stage_v, out_hbm.at[c, pl.ds(s * TPN + t * TCH, TCH)])


def _post_body(u0_ref, a_ref, e_ref, wr1_ref, br1_ref, wr2_ref, br2_ref,
               wv_ref, bv_ref, gate_ref, out_ref):
    u = u0_ref[...] + a_ref[0] + a_ref[1]
    for k in range(NR):
        h = _sp(u)
        h = _sp(jnp.dot(h, wr1_ref[k], preferred_element_type=jnp.float32) + br1_ref[k])
        h = jnp.dot(h, wr2_ref[k], preferred_element_type=jnp.float32) + br2_ref[k]
        u = u + h
    u = _sp(u)
    out_ref[...] = (gate_ref[...] * e_ref[...]
                    + jnp.dot(u, wv_ref[...], preferred_element_type=jnp.float32)
                    + bv_ref[...])


_post = pl.pallas_call(
    _post_body,
    grid=(N // NB,),
    in_specs=[
        pl.BlockSpec((NB, F), lambda i: (i, 0)),
        pl.BlockSpec((NC, NB, F), lambda i: (0, i, 0)),
        pl.BlockSpec((NB, F), lambda i: (i, 0)),
        pl.BlockSpec((NR, F, F), lambda i: (0, 0, 0)),
        pl.BlockSpec((NR, 1, F), lambda i: (0, 0, 0)),
        pl.BlockSpec((NR, F, F), lambda i: (0, 0, 0)),
        pl.BlockSpec((NR, 1, F), lambda i: (0, 0, 0)),
        pl.BlockSpec((F, F), lambda i: (0, 0)),
        pl.BlockSpec((1, F), lambda i: (0, 0)),
        pl.BlockSpec((1, F), lambda i: (0, 0)),
    ],
    out_specs=pl.BlockSpec((NB, F), lambda i: (i, 0)),
    out_shape=jax.ShapeDtypeStruct((N, F), jnp.float32),
)


def kernel(pair_indices, atomic_embedding, f_ij, W_g, W_i, b_i, W_j, b_j,
           W_v, b_v, Wr1, br1, Wr2, br2, gate):
    idx = pair_indices.astype(jnp.int32)
    idx_i = idx[0]
    idx_j = idx[1]

    e, u0, ej = _node_pre(atomic_embedding, W_i, b_i.reshape(1, F), W_j, b_j.reshape(1, F))
    g = _g_mat(f_ij, W_g)
    accs = _edge_kernel(idx_i, idx_j, g, ej)
    out = _post(u0, accs, e, Wr1, br1.reshape(NR, 1, F), Wr2, br2.reshape(NR, 1, F),
                W_v, b_v.reshape(1, F), gate.reshape(1, F))
    return out


# speedup vs baseline: 3.5846x; 1.2922x over previous
import functools

import numpy as np
import jax
import jax.numpy as jnp
from jax import lax
from jax.experimental import pallas as pl
from jax.experimental.pallas import tpu as pltpu
from jax.experimental.pallas import tpu_sc as plsc

N = 10000
E = 320000
F = 128
R = 32
NR = 3

NB = 1000
EB = 6400

NC = 2
NS = 16
L = 16
NW = NC * NS
EPW = E // NW
C = 80
NCHUNK = EPW // C
NPAD = 10240
TPN = NPAD // NS
TCH = 32


def _sp(x):
    return jnp.maximum(x, 0.0) + jnp.log1p(jnp.exp(-jnp.abs(x)))


def _node_pre_body(x_ref, wi_ref, bi_ref, wj_ref, bj_ref, e_ref, u0_ref, ej_ref):
    e = _sp(x_ref[...])
    e_ref[...] = e
    u0_ref[...] = _sp(jnp.dot(e, wi_ref[...], preferred_element_type=jnp.float32) + bi_ref[...])
    ej_ref[...] = _sp(jnp.dot(e, wj_ref[...], preferred_element_type=jnp.float32) + bj_ref[...])


_node_pre = pl.pallas_call(
    _node_pre_body,
    grid=(N // NB,),
    in_specs=[
        pl.BlockSpec((NB, F), lambda i: (i, 0)),
        pl.BlockSpec((F, F), lambda i: (0, 0)),
        pl.BlockSpec((1, F), lambda i: (0, 0)),
        pl.BlockSpec((F, F), lambda i: (0, 0)),
        pl.BlockSpec((1, F), lambda i: (0, 0)),
    ],
    out_specs=[pl.BlockSpec((NB, F), lambda i: (i, 0))] * 3,
    out_shape=[jax.ShapeDtypeStruct((N, F), jnp.float32)] * 3,
)


def _g_body(f_ref, wg_ref, g_ref):
    x = jax.lax.dot_general(f_ref[...], wg_ref[...], (((0,), (0,)), ((), ())),
                            preferred_element_type=jnp.float32)
    y = jax.lax.bitcast_convert_type(x, jnp.int32)
    r = y + jnp.int32(0x7FFF) + jax.lax.bitwise_and(
        jax.lax.shift_right_logical(y, 16), jnp.int32(1))
    lo = jax.lax.bitwise_and(jax.lax.shift_right_logical(r[:, :F // 2], 16),
                             jnp.int32(0xFFFF))
    hi = jax.lax.bitwise_and(r[:, F // 2:], jnp.int32(-65536))
    g_ref[...] = jax.lax.bitwise_or(lo, hi)


_g_mat = pl.pallas_call(
    _g_body,
    grid=(E // EB,),
    in_specs=[
        pl.BlockSpec((R, EB), lambda i: (0, i)),
        pl.BlockSpec((R, F), lambda i: (0, 0)),
    ],
    out_specs=pl.BlockSpec((EB, F // 2), lambda i: (i, 0)),
    out_shape=jax.ShapeDtypeStruct((E, F // 2), jnp.int32),
)


@functools.partial(
    pl.kernel,
    out_type=jax.ShapeDtypeStruct((NC, NPAD, F), jnp.float32),
    mesh=plsc.VectorSubcoreMesh(core_axis_name="c", subcore_axis_name="s"),
    scratch_types=[
        pltpu.VMEM((C,), jnp.int32),
        pltpu.VMEM((C,), jnp.int32),
        pltpu.VMEM((C,), jnp.int32),
        pltpu.VMEM((C,), jnp.int32),
        pltpu.VMEM((C, F), jnp.float32),
        pltpu.VMEM((C, F), jnp.float32),
        pltpu.VMEM((C, F // 2), jnp.int32),
        pltpu.VMEM((C, F // 2), jnp.int32),
        pltpu.VMEM((C,), jnp.int32),
        pltpu.VMEM((C,), jnp.int32),
        pltpu.VMEM((TCH, F), jnp.float32),
        pltpu.VMEM_SHARED((NPAD, F), jnp.float32),
        pltpu.SemaphoreType.DMA,
        pltpu.SemaphoreType.DMA,
        pltpu.SemaphoreType.DMA,
        pltpu.SemaphoreType.DMA,
        pltpu.SemaphoreType.DMA,
        pltpu.SemaphoreType.DMA,
        pltpu.SemaphoreType.DMA,
        pltpu.SemaphoreType.DMA,
        pltpu.SemaphoreType.DMA,
        pltpu.SemaphoreType.DMA,
    ],
)
def _edge_kernel(ii_hbm, jj_hbm, g_hbm, ej_hbm, out_hbm,
                 ii0, ii1, jj0, jj1, rows0, rows1, g0, g1, sii0, sii1,
                 stage_v, acc_sh,
                 sem_ii0, sem_ii1, sem_jj0, sem_jj1, sem_ga0, sem_ga1,
                 sem_gl0, sem_gl1, sem_s0, sem_s1):
    iiv = (ii0, ii1)
    jjv = (jj0, jj1)
    rows = (rows0, rows1)
    gv = (g0, g1)
    sii = (sii0, sii1)
    sem_ii = (sem_ii0, sem_ii1)
    sem_jj = (sem_jj0, sem_jj1)
    sem_ga = (sem_ga0, sem_ga1)
    sem_gl = (sem_gl0, sem_gl1)
    sem_s = (sem_s0, sem_s1)

    c = lax.axis_index("c")
    s = lax.axis_index("s")
    wid = s * NC + c
    base_w = wid * EPW

    zv = jnp.zeros((L,), jnp.float32)

    def _zrow(i, cc):
        for v in range(F // L):
            stage_v[i, pl.ds(v * L, L)] = zv
        return cc

    lax.fori_loop(0, TCH, _zrow, 0)
    for t in range(TPN // TCH):
        pltpu.sync_copy(stage_v, acc_sh.at[pl.ds(s * TPN + t * TCH, TCH)])
    plsc.subcore_barrier()

    def start_I(k, b):
        base = base_w + k * C
        pltpu.async_copy(ii_hbm.at[pl.ds(base, C)], iiv[b], sem_ii[b])
        pltpu.async_copy(jj_hbm.at[pl.ds(base, C)], jjv[b], sem_jj[b])

    def wait_ii(b):
        pltpu.make_async_copy(ii_hbm.at[pl.ds(0, C)], iiv[b], sem_ii[b]).wait()

    def wait_jj(b):
        pltpu.make_async_copy(jj_hbm.at[pl.ds(0, C)], jjv[b], sem_jj[b]).wait()

    def start_G(k, b):
        base = base_w + k * C
        pltpu.async_copy(ej_hbm.at[jjv[b]], rows[b], sem_ga[b])
        pltpu.async_copy(g_hbm.at[pl.ds(base, C)], gv[b], sem_gl[b])

    def wait_G(b):
        pltpu.make_async_copy(ej_hbm.at[pl.ds(0, C)], rows[b], sem_ga[b]).wait()
        pltpu.make_async_copy(g_hbm.at[pl.ds(0, C)], gv[b], sem_gl[b]).wait()

    def copy_sii(b):
        for v in range(C // L):
            sii[b][pl.ds(v * L, L)] = iiv[b][pl.ds(v * L, L)]

    def mul(b):
        def _mrow(i, cc):
            for v in range(F // 32):
                gi = gv[b][i, pl.ds(L * v, L)]
                glo = jax.lax.bitcast_convert_type(
                    jax.lax.shift_left(gi, 16), jnp.float32)
                ghi = jax.lax.bitcast_convert_type(
                    jax.lax.bitwise_and(gi, jnp.int32(-65536)), jnp.float32)
                ix_lo = (i, pl.ds(L * v, L))
                ix_hi = (i, pl.ds(F // 2 + L * v, L))
                rows[b][ix_lo] = rows[b][ix_lo] * glo
                rows[b][ix_hi] = rows[b][ix_hi] * ghi
            return cc

        lax.fori_loop(0, C, _mrow, 0)

    def start_S(b):
        pltpu.async_copy(rows[b], acc_sh.at[sii[b]], sem_s[b], add=True)

    def wait_S(b):
        pltpu.make_async_copy(ej_hbm.at[pl.ds(0, C)], rows[b], sem_s[b]).wait()

    def step(k, b):
        wait_G(b)
        wait_ii(b)
        copy_sii(b)
        start_I(k + 2, b)
        wait_S(1 - b)
        wait_jj(1 - b)
        start_G(k + 1, 1 - b)
        mul(b)
        start_S(b)

    start_I(0, 0)
    start_I(1, 1)
    wait_jj(0)
    start_G(0, 0)
    wait_G(0)
    wait_ii(0)
    copy_sii(0)
    start_I(2, 0)
    wait_jj(1)
    start_G(1, 1)
    mul(0)
    start_S(0)

    def _pair(j2, cc):
        k = 1 + 2 * j2
        step(k, 1)
        step(k + 1, 0)
        return cc

    lax.fori_loop(0, (NCHUNK - 3) // 2, _pair, 0)

    wait_G(1)
    wait_ii(1)
    copy_sii(1)
    wait_S(0)
    wait_jj(0)
    start_G(NCHUNK - 1, 0)
    mul(1)
    start_S(1)

    wait_G(0)
    wait_ii(0)
    wait_S(1)
    mul(0)
    pltpu.sync_copy(rows0, acc_sh.at[ii0], add=True)

    plsc.subcore_barrier()

    for t in range(TPN // TCH):
        pltpu.sync_copy(acc_sh.at[pl.ds(s * TPN + t * TCH, TCH)], stage_v)
        pltpu.sync_copy(stage_v, out_hbm.at[c, pl.ds(s * TPN + t * TCH, TCH)])


def _post_body(u0_ref, a_ref, e_ref, wr1_ref, br1_ref, wr2_ref, br2_ref,
               wv_ref, bv_ref, gate_ref, out_ref):
    u = u0_ref[...] + a_ref[0] + a_ref[1]
    for k in range(NR):
        h = _sp(u)
        h = _sp(jnp.dot(h, wr1_ref[k], preferred_element_type=jnp.float32) + br1_ref[k])
        h = jnp.dot(h, wr2_ref[k], preferred_element_type=jnp.float32) + br2_ref[k]
        u = u + h
    u = _sp(u)
    out_ref[...] = (gate_ref[...] * e_ref[...]
                    + jnp.dot(u, wv_ref[...], preferred_element_type=jnp.float32)
                    + bv_ref[...])


_post = pl.pallas_call(
    _post_body,
    grid=(N // NB,),
    in_specs=[
        pl.BlockSpec((NB, F), lambda i: (i, 0)),
        pl.BlockSpec((NC, NB, F), lambda i: (0, i, 0)),
        pl.BlockSpec((NB, F), lambda i: (i, 0)),
        pl.BlockSpec((NR, F, F), lambda i: (0, 0, 0)),
        pl.BlockSpec((NR, 1, F), lambda i: (0, 0, 0)),
        pl.BlockSpec((NR, F, F), lambda i: (0, 0, 0)),
        pl.BlockSpec((NR, 1, F), lambda i: (0, 0, 0)),
        pl.BlockSpec((F, F), lambda i: (0, 0)),
        pl.BlockSpec((1, F), lambda i: (0, 0)),
        pl.BlockSpec((1, F), lambda i: (0, 0)),
    ],
    out_specs=pl.BlockSpec((NB, F), lambda i: (i, 0)),
    out_shape=jax.ShapeDtypeStruct((N, F), jnp.float32),
)


def kernel(pair_indices, atomic_embedding, f_ij, W_g, W_i, b_i, W_j, b_j,
           W_v, b_v, Wr1, br1, Wr2, br2, gate):
    idx = pair_indices.astype(jnp.int32)
    idx_i = idx[0]
    idx_j = idx[1]

    e, u0, ej = _node_pre(atomic_embedding, W_i, b_i.reshape(1, F), W_j, b_j.reshape(1, F))
    g = _g_mat(f_ij.T, W_g)
    accs = _edge_kernel(idx_i, idx_j, g, ej)
    out = _post(u0, accs, e, Wr1, br1.reshape(NR, 1, F), Wr2, br2.reshape(NR, 1, F),
                W_v, b_v.reshape(1, F), gate.reshape(1, F))
    return out
